# Initial kernel scaffold; baseline (speedup 1.0000x reference)
import jax
import jax.numpy as jnp
from jax.experimental import pallas as pl


def kernel(inputs):
    raise NotImplementedError("write your pallas kernel here")


# speedup vs baseline: 8.4375x; 8.4375x over previous
import functools

import jax
import jax.numpy as jnp
from jax.experimental import pallas as pl
from jax.experimental.pallas import tpu as pltpu

K = 20


def _edge_kernel(x_blk_ref, x_all_ref, out_ref, *, n, d, k):
    x = x_blk_ref[0]
    xa = x_all_ref[0]

    inner = jnp.dot(x, xa.T, preferred_element_type=jnp.float32)
    xsq = jnp.sum(x * x, axis=1, keepdims=True)
    xasq = jnp.sum(xa * xa, axis=1, keepdims=True).T
    dist = xsq + (-2.0 * inner) + xasq

    iota = jax.lax.broadcasted_iota(jnp.int32, dist.shape, 1)
    neighbors = []
    for _ in range(k):
        m = jnp.min(dist, axis=1, keepdims=True)
        eq = dist == m
        first = jnp.min(jnp.where(eq, iota, n), axis=1, keepdims=True)
        sel = iota == first
        onehot = sel.astype(jnp.float32)
        neighbors.append(jnp.dot(onehot, xa, preferred_element_type=jnp.float32))
        dist = jnp.where(sel, jnp.inf, dist)

    for j in range(k):
        base = j * 2 * d
        out_ref[0, :, base:base + d] = x
        out_ref[0, :, base + d:base + 2 * d] = neighbors[j] - x


def kernel(inputs):
    b, n, d = inputs.shape
    block = 256
    grid = (b, n // block)

    out = pl.pallas_call(
        functools.partial(_edge_kernel, n=n, d=d, k=K),
        grid=grid,
        in_specs=[
            pl.BlockSpec((1, block, d), lambda i, j: (i, j, 0)),
            pl.BlockSpec((1, n, d), lambda i, j: (i, 0, 0)),
        ],
        out_specs=pl.BlockSpec((1, block, 2 * d * K), lambda i, j: (i, j, 0)),
        out_shape=jax.ShapeDtypeStruct((b, n, 2 * d * K), jnp.float32),
    )(inputs, inputs)
    return out.reshape(b, n, K, 2 * d)


# speedup vs baseline: 9.2217x; 1.0929x over previous
import functools

import jax
import jax.numpy as jnp
from jax.experimental import pallas as pl
from jax.experimental.pallas import tpu as pltpu

K = 20


def _edge_kernel(x_blk_ref, x_all_ref, out_ref, *, n, d, k):
    x = x_blk_ref[0]
    xa = x_all_ref[0]

    inner = jnp.dot(x, xa.T, preferred_element_type=jnp.float32)
    xsq = jnp.sum(x * x, axis=1, keepdims=True)
    xasq = jnp.sum(xa * xa, axis=1, keepdims=True).T
    dist = xsq + (-2.0 * inner) + xasq

    iota = jax.lax.broadcasted_iota(
        jnp.int32, dist.shape, 1).astype(jnp.float32)
    nf = jnp.float32(n)
    neighbors = []
    for _ in range(k):
        m = jnp.min(dist, axis=1, keepdims=True)
        eq = dist == m
        first = jnp.min(jnp.where(eq, iota, nf), axis=1, keepdims=True)
        sel = iota == first
        onehot = sel.astype(jnp.float32)
        neighbors.append(jnp.dot(onehot, xa, preferred_element_type=jnp.float32))
        dist = jnp.where(sel, jnp.inf, dist)

    for j in range(k):
        base = j * 2 * d
        out_ref[0, :, base:base + d] = x
        out_ref[0, :, base + d:base + 2 * d] = neighbors[j] - x


def kernel(inputs):
    b, n, d = inputs.shape
    block = 512
    grid = (b, n // block)

    out = pl.pallas_call(
        functools.partial(_edge_kernel, n=n, d=d, k=K),
        grid=grid,
        in_specs=[
            pl.BlockSpec((1, block, d), lambda i, j: (i, j, 0)),
            pl.BlockSpec((1, n, d), lambda i, j: (i, 0, 0)),
        ],
        out_specs=pl.BlockSpec((1, block, 2 * d * K), lambda i, j: (i, j, 0)),
        out_shape=jax.ShapeDtypeStruct((b, n, 2 * d * K), jnp.float32),
    )(inputs, inputs)
    return out.reshape(b, n, K, 2 * d)
